# baseline (speedup 1.0000x reference)
import jax
import jax.numpy as jnp
from jax.experimental import pallas as pl
from jax.experimental.pallas import tpu as pltpu

_B = 128
_N = 32768
_CW = 16384
_NC = _N // _CW
_NB = 2
_DEPTH = 2


def _body(x_hbm, o_hbm, xb, sx, so):

    def in_copy(c):
        slot = c % _NB
        return pltpu.make_async_copy(
            x_hbm.at[:, pl.ds(c * _CW, _CW)], xb.at[slot], sx.at[slot])

    def out_copy(c):
        slot = c % _NB
        return pltpu.make_async_copy(
            xb.at[slot], o_hbm.at[:, pl.ds(c * _CW, _CW)], so.at[slot])

    for c in range(_DEPTH):
        in_copy(c).start()

    for c in range(_NC):
        in_copy(c).wait()
        if c >= _NB:
            out_copy(c - _NB).wait()
        out_copy(c).start()
        if c + _DEPTH < _NC:
            in_copy(c + _DEPTH).start()

    for c in range(max(_NC - _NB, 0), _NC):
        out_copy(c).wait()


def _body2(x_hbm, m_hbm, o_hbm, xb, sx, so):
    _body(x_hbm, o_hbm, xb, sx, so)


def kernel(x, mask):
    shifts = jnp.arange(32, dtype=jnp.uint32)[None, :, None]
    m32 = jnp.sum(
        mask.reshape(_B, 32, _N // 32).astype(jnp.uint32) << shifts, axis=1)
    return pl.pallas_call(
        _body2,
        in_specs=[
            pl.BlockSpec(memory_space=pltpu.MemorySpace.HBM),
            pl.BlockSpec(memory_space=pltpu.MemorySpace.HBM),
        ],
        out_specs=pl.BlockSpec(memory_space=pltpu.MemorySpace.HBM),
        out_shape=jax.ShapeDtypeStruct((_B, _N), jnp.float32),
        scratch_shapes=[
            pltpu.VMEM((_NB, _B, _CW), jnp.float32),
            pltpu.SemaphoreType.DMA((_NB,)),
            pltpu.SemaphoreType.DMA((_NB,)),
        ],
    )(x, m32)


# speedup vs baseline: 3.0960x; 3.0960x over previous
import jax
import jax.numpy as jnp
from jax.experimental import pallas as pl
from jax.experimental.pallas import tpu as pltpu

_B = 128
_N = 32768
_CW = 8192
_NC = _N // _CW
_NB = 4
_DEPTH = 3


def _body(x_hbm, m8_hbm, o_hbm, xb, mb, ob, sx, sm, so):

    def in_copies(c):
        slot = c % _NB
        cx = pltpu.make_async_copy(
            x_hbm.at[:, pl.ds(c * _CW, _CW)], xb.at[slot], sx.at[slot])
        cm = pltpu.make_async_copy(
            m8_hbm.at[:, pl.ds(c * _CW, _CW)], mb.at[slot], sm.at[slot])
        return cx, cm

    def out_copy(c):
        slot = c % _NB
        return pltpu.make_async_copy(
            ob.at[slot], o_hbm.at[:, pl.ds(c * _CW, _CW)], so.at[slot])

    for c in range(min(_DEPTH, _NC)):
        cx, cm = in_copies(c)
        cx.start()
        cm.start()

    for c in range(_NC):
        slot = c % _NB
        cx, cm = in_copies(c)
        cx.wait()
        cm.wait()
        if c >= _NB:
            out_copy(c - _NB).wait()
        ob[slot] = jnp.where(mb[slot] != 0, xb[slot], 0.0)
        out_copy(c).start()
        if c + _DEPTH < _NC:
            nx, nm = in_copies(c + _DEPTH)
            nx.start()
            nm.start()

    for c in range(max(_NC - _NB, 0), _NC):
        out_copy(c).wait()


def kernel(x, mask):
    mask8 = mask.view(jnp.int8)
    return pl.pallas_call(
        _body,
        in_specs=[
            pl.BlockSpec(memory_space=pltpu.MemorySpace.HBM),
            pl.BlockSpec(memory_space=pltpu.MemorySpace.HBM),
        ],
        out_specs=pl.BlockSpec(memory_space=pltpu.MemorySpace.HBM),
        out_shape=jax.ShapeDtypeStruct((_B, _N), jnp.float32),
        scratch_shapes=[
            pltpu.VMEM((_NB, _B, _CW), jnp.float32),
            pltpu.VMEM((_NB, _B, _CW), jnp.int8),
            pltpu.VMEM((_NB, _B, _CW), jnp.float32),
            pltpu.SemaphoreType.DMA((_NB,)),
            pltpu.SemaphoreType.DMA((_NB,)),
            pltpu.SemaphoreType.DMA((_NB,)),
        ],
    )(x, mask8)


# speedup vs baseline: 3.1774x; 1.0263x over previous
import jax
import jax.numpy as jnp
from jax.experimental import pallas as pl
from jax.experimental.pallas import tpu as pltpu

_B = 128
_N = 32768
_CW = 16384
_NC = _N // _CW
_NB = 2
_DEPTH = 2


def _body(x_hbm, m8_hbm, o_hbm, xb, mb, ob, sx, sm, so):

    def in_copies(c):
        slot = c % _NB
        cx = pltpu.make_async_copy(
            x_hbm.at[:, pl.ds(c * _CW, _CW)], xb.at[slot], sx.at[slot])
        cm = pltpu.make_async_copy(
            m8_hbm.at[:, pl.ds(c * _CW, _CW)], mb.at[slot], sm.at[slot])
        return cx, cm

    def out_copy(c):
        slot = c % _NB
        return pltpu.make_async_copy(
            ob.at[slot], o_hbm.at[:, pl.ds(c * _CW, _CW)], so.at[slot])

    for c in range(min(_DEPTH, _NC)):
        cx, cm = in_copies(c)
        cx.start()
        cm.start()

    for c in range(_NC):
        slot = c % _NB
        cx, cm = in_copies(c)
        cx.wait()
        cm.wait()
        if c >= _NB:
            out_copy(c - _NB).wait()
        ob[slot] = jnp.where(mb[slot] != 0, xb[slot], 0.0)
        out_copy(c).start()
        if c + _DEPTH < _NC:
            nx, nm = in_copies(c + _DEPTH)
            nx.start()
            nm.start()

    for c in range(max(_NC - _NB, 0), _NC):
        out_copy(c).wait()


def kernel(x, mask):
    mask8 = mask.view(jnp.int8)
    return pl.pallas_call(
        _body,
        in_specs=[
            pl.BlockSpec(memory_space=pltpu.MemorySpace.HBM),
            pl.BlockSpec(memory_space=pltpu.MemorySpace.HBM),
        ],
        out_specs=pl.BlockSpec(memory_space=pltpu.MemorySpace.HBM),
        out_shape=jax.ShapeDtypeStruct((_B, _N), jnp.float32),
        scratch_shapes=[
            pltpu.VMEM((_NB, _B, _CW), jnp.float32),
            pltpu.VMEM((_NB, _B, _CW), jnp.int8),
            pltpu.VMEM((_NB, _B, _CW), jnp.float32),
            pltpu.SemaphoreType.DMA((_NB,)),
            pltpu.SemaphoreType.DMA((_NB,)),
            pltpu.SemaphoreType.DMA((_NB,)),
        ],
    )(x, mask8)
